# baseline (speedup 1.0000x reference)
import functools

import jax
import jax.numpy as jnp
from jax.experimental import pallas as pl
from jax.experimental.pallas import tpu as pltpu

_STAGE = 16


def _pos_kernel(row_ref, col_ref, out_ref, scratch, sem, *, b):
    k, rows, lanes = scratch.shape
    v = row_ref[0, 0]
    scratch[:] = jnp.full((k, rows, lanes), v, jnp.float32)
    n_dma = b // k
    for i in range(n_dma):
        pltpu.make_async_copy(
            scratch, out_ref.at[pl.ds(i * k, k)], sem).start()
    for i in range(n_dma):
        pltpu.make_async_copy(
            scratch, out_ref.at[pl.ds(i * k, k)], sem).wait()


def kernel(x, row_embed, col_embed):
    b = x.shape[0]
    h, w = x.shape[2], x.shape[3]
    n, d = row_embed.shape
    hw = h * w
    rows = (2 * d * hw) // 128
    body = functools.partial(_pos_kernel, b=b)
    out = pl.pallas_call(
        body,
        in_specs=[
            pl.BlockSpec((n, d), lambda: (0, 0)),
            pl.BlockSpec((n, d), lambda: (0, 0)),
        ],
        out_specs=pl.BlockSpec(memory_space=pltpu.MemorySpace.HBM),
        out_shape=jax.ShapeDtypeStruct((b, rows, 128), jnp.float32),
        scratch_shapes=[
            pltpu.VMEM((_STAGE, rows, 128), jnp.float32),
            pltpu.SemaphoreType.DMA,
        ],
    )(row_embed, col_embed)
    return out.reshape(b, 2 * d, h, w)


# speedup vs baseline: 3.1050x; 3.1050x over previous
import functools

import jax
import jax.numpy as jnp
from jax.experimental import pallas as pl
from jax.experimental.pallas import tpu as pltpu

_BBLK = 4


def _pos_kernel(row_ref, col_ref, out_ref, scratch, *, h, w, d):
    hw = h * w

    @pl.when(pl.program_id(0) == 0)
    def _():
        p = jax.lax.broadcasted_iota(jnp.int32, (max(h, w), hw), 1)
        q = jax.lax.broadcasted_iota(jnp.int32, (max(h, w), hw), 0)
        sel_col = (p % w == q).astype(jnp.float32)[:w, :]
        sel_row = (p // w == q).astype(jnp.float32)[:h, :]
        scratch[:d, :] = jax.lax.dot_general(
            col_ref[:w, :], sel_col, (((0,), (0,)), ((), ())),
            preferred_element_type=jnp.float32,
            precision=jax.lax.Precision.HIGHEST)
        scratch[d:, :] = jax.lax.dot_general(
            row_ref[:h, :], sel_row, (((0,), (0,)), ((), ())),
            preferred_element_type=jnp.float32,
            precision=jax.lax.Precision.HIGHEST)

    out_ref[...] = jnp.broadcast_to(scratch[...][None], out_ref.shape)


def kernel(x, row_embed, col_embed):
    b = x.shape[0]
    h, w = x.shape[2], x.shape[3]
    n, d = row_embed.shape
    body = functools.partial(_pos_kernel, h=h, w=w, d=d)
    out = pl.pallas_call(
        body,
        grid=(b // _BBLK,),
        in_specs=[
            pl.BlockSpec((n, d), lambda i: (0, 0)),
            pl.BlockSpec((n, d), lambda i: (0, 0)),
        ],
        out_specs=pl.BlockSpec((_BBLK, 2 * d, h * w), lambda i: (i, 0, 0)),
        out_shape=jax.ShapeDtypeStruct((b, 2 * d, h * w), jnp.float32),
        scratch_shapes=[
            pltpu.VMEM((2 * d, h * w), jnp.float32),
        ],
    )(row_embed, col_embed)
    return out.reshape(b, 2 * d, h, w)


# speedup vs baseline: 11.7686x; 3.7902x over previous
import functools

import jax
import jax.numpy as jnp
from jax.experimental import pallas as pl

_PBLK = 28


def _pos_kernel(row_ref, col_ref, out_ref, *, h, w, d):
    blk = out_ref.shape[0]
    b = out_ref.shape[1]
    p = _PBLK * pl.program_id(0) + jax.lax.broadcasted_iota(
        jnp.int32, (blk, max(h, w)), 0)
    q = jax.lax.broadcasted_iota(jnp.int32, (blk, max(h, w)), 1)
    sel_col = (p % w == q).astype(jnp.float32)[:, :w]
    sel_row = (p // w == q).astype(jnp.float32)[:, :h]
    top = jax.lax.dot_general(
        sel_col, col_ref[:w, :], (((1,), (0,)), ((), ())),
        preferred_element_type=jnp.float32,
        precision=jax.lax.Precision.HIGHEST)
    bottom = jax.lax.dot_general(
        sel_row, row_ref[:h, :], (((1,), (0,)), ((), ())),
        preferred_element_type=jnp.float32,
        precision=jax.lax.Precision.HIGHEST)
    vec = jnp.concatenate([top, bottom], axis=1)
    out_ref[...] = jnp.broadcast_to(vec[:, None, :], (blk, b, 2 * d))


def kernel(x, row_embed, col_embed):
    b = x.shape[0]
    h, w = x.shape[2], x.shape[3]
    n, d = row_embed.shape
    hw = h * w
    body = functools.partial(_pos_kernel, h=h, w=w, d=d)
    out = pl.pallas_call(
        body,
        grid=(hw // _PBLK,),
        in_specs=[
            pl.BlockSpec((n, d), lambda i: (0, 0)),
            pl.BlockSpec((n, d), lambda i: (0, 0)),
        ],
        out_specs=pl.BlockSpec((_PBLK, b, 2 * d), lambda i: (i, 0, 0)),
        out_shape=jax.ShapeDtypeStruct((hw, b, 2 * d), jnp.float32),
    )(row_embed, col_embed)
    return jnp.transpose(out.reshape(h, w, b, 2 * d), (2, 3, 0, 1))


# speedup vs baseline: 13.5934x; 1.1551x over previous
import functools

import jax
import jax.numpy as jnp
from jax.experimental import pallas as pl

_PBLK = 49


def _pos_kernel(row_ref, col_ref, out_ref, *, h, w, d):
    blk = out_ref.shape[0]
    b = out_ref.shape[1]
    p = _PBLK * pl.program_id(0) + jax.lax.broadcasted_iota(
        jnp.int32, (blk, max(h, w)), 0)
    q = jax.lax.broadcasted_iota(jnp.int32, (blk, max(h, w)), 1)
    sel_col = (p % w == q).astype(jnp.float32)[:, :w]
    sel_row = (p // w == q).astype(jnp.float32)[:, :h]
    top = jax.lax.dot_general(
        sel_col, col_ref[:w, :], (((1,), (0,)), ((), ())),
        preferred_element_type=jnp.float32,
        precision=jax.lax.Precision.HIGHEST)
    bottom = jax.lax.dot_general(
        sel_row, row_ref[:h, :], (((1,), (0,)), ((), ())),
        preferred_element_type=jnp.float32,
        precision=jax.lax.Precision.HIGHEST)
    vec = jnp.concatenate([top, bottom], axis=1)
    out_ref[...] = jnp.broadcast_to(vec[:, None, :], (blk, b, 2 * d))


def kernel(x, row_embed, col_embed):
    b = x.shape[0]
    h, w = x.shape[2], x.shape[3]
    n, d = row_embed.shape
    hw = h * w
    body = functools.partial(_pos_kernel, h=h, w=w, d=d)
    out = pl.pallas_call(
        body,
        grid=(hw // _PBLK,),
        in_specs=[
            pl.BlockSpec((n, d), lambda i: (0, 0)),
            pl.BlockSpec((n, d), lambda i: (0, 0)),
        ],
        out_specs=pl.BlockSpec((_PBLK, b, 2 * d), lambda i: (i, 0, 0)),
        out_shape=jax.ShapeDtypeStruct((hw, b, 2 * d), jnp.float32),
    )(row_embed, col_embed)
    return jnp.transpose(out.reshape(h, w, b, 2 * d), (2, 3, 0, 1))
